# Initial kernel scaffold; baseline (speedup 1.0000x reference)
import jax
import jax.numpy as jnp
from jax.experimental import pallas as pl


def kernel(target_embeddings, target_slice_idx, cluster_centers):
    raise NotImplementedError("write your pallas kernel here")


# speedup vs baseline: 1.1091x; 1.1091x over previous
import functools

import jax
import jax.numpy as jnp
from jax import lax
from jax.experimental import pallas as pl
from jax.experimental.pallas import tpu as pltpu

_BQ = 512
_BK = 1024


def _nn_loss_kernel(a_ref, b_ref, out_ref, acc_ref, *, inv_q):
    i = pl.program_id(0)
    j = pl.program_id(1)
    nq = pl.num_programs(0)
    nk = pl.num_programs(1)

    a = a_ref[...]
    b = b_ref[...]
    g = jnp.dot(a, b, preferred_element_type=jnp.float32)
    b2 = jnp.sum(b * b, axis=0)
    tile_min = jnp.min(b2[None, :] - 2.0 * g, axis=1, keepdims=True)

    @pl.when(j == 0)
    def _():
        acc_ref[...] = tile_min

    @pl.when(j > 0)
    def _():
        acc_ref[...] = jnp.minimum(acc_ref[...], tile_min)

    @pl.when(j == nk - 1)
    def _():
        a2 = jnp.sum(a * a, axis=1, keepdims=True)
        d2 = a2 + acc_ref[...]
        psum = jnp.sum(jnp.sqrt(jnp.maximum(d2, 1e-12))).reshape(1, 1)
        tot = jnp.where(i == 0, psum, out_ref[...] + psum)
        out_ref[...] = jnp.where(i == nq - 1, tot * inv_q, tot)


@jax.jit
def kernel(target_embeddings, target_slice_idx, cluster_centers):
    del target_slice_idx
    q, d = target_embeddings.shape
    k = cluster_centers.shape[0]
    centers_t = cluster_centers.T

    out = pl.pallas_call(
        functools.partial(_nn_loss_kernel, inv_q=1.0 / q),
        grid=(q // _BQ, k // _BK),
        in_specs=[
            pl.BlockSpec((_BQ, d), lambda i, j: (i, 0)),
            pl.BlockSpec((d, _BK), lambda i, j: (0, j)),
        ],
        out_specs=pl.BlockSpec((1, 1), lambda i, j: (0, 0)),
        out_shape=jax.ShapeDtypeStruct((1, 1), jnp.float32),
        scratch_shapes=[pltpu.VMEM((_BQ, 1), jnp.float32)],
    )(target_embeddings, centers_t)
    return out[0, 0]
